# baseline (speedup 1.0000x reference)
import functools

import jax
import jax.numpy as jnp
from jax import lax
from jax.experimental import pallas as pl
from jax.experimental.pallas import tpu as pltpu
from jax.experimental.pallas import tpu_sc as plsc

_NC = 2
_NS = 16
_NW = _NC * _NS


def _sc_fused(xidx, word_table, pos_table, batch, seq):
    V, D = word_table.shape
    B = batch * seq
    p_per_w = seq // _NW
    PC = 16
    n_chunks = p_per_w // PC
    rows = batch * PC
    factor = float(D) ** 0.5
    nvec = D // 16

    mesh = plsc.VectorSubcoreMesh(core_axis_name="c", subcore_axis_name="s")

    @functools.partial(
        pl.kernel,
        mesh=mesh,
        out_type=[
            jax.ShapeDtypeStruct((B, D), jnp.float32),
            jax.ShapeDtypeStruct((_NW, n_chunks, rows), jnp.int32),
        ],
        scratch_types=[
            pltpu.VMEM((n_chunks, rows), jnp.int32),
            pltpu.VMEM((n_chunks, rows), jnp.int32),
            pltpu.VMEM((rows, D), jnp.float32),
            pltpu.VMEM((rows, D), jnp.float32),
            pltpu.VMEM((PC, D), jnp.float32),
            pltpu.VMEM((PC, D), jnp.float32),
            pltpu.SemaphoreType.DMA,
            pltpu.SemaphoreType.DMA,
            pltpu.SemaphoreType.DMA,
            pltpu.SemaphoreType.DMA,
        ],
    )
    def k(x_hbm, wt_hbm, pos_hbm, out_hbm, mask_hbm,
          idx_v, mask_v, gbuf0, gbuf1, pbuf0, pbuf1,
          gsem0, gsem1, psem0, psem1):
        wid = lax.axis_index("s") * _NC + lax.axis_index("c")
        pbase = wid * p_per_w
        gbufs = (gbuf0, gbuf1)
        pbufs = (pbuf0, pbuf1)
        gsems = (gsem0, gsem1)
        psems = (psem0, psem1)

        pltpu.sync_copy(x_hbm.at[wid], idx_v)

        for c in range(n_chunks):
            for g in range(rows // 16):
                v = idx_v[c, pl.ds(g * 16, 16)]
                mask_v[c, pl.ds(g * 16, 16)] = jnp.where(
                    v == 0, jnp.full((16,), 1, jnp.int32),
                    jnp.full((16,), 0, jnp.int32))
        pltpu.sync_copy(mask_v, mask_hbm.at[wid])

        def start_chunk(c):
            gh = pltpu.async_copy(
                wt_hbm.at[idx_v.at[c]], gbufs[c % 2], gsems[c % 2])
            ph = pltpu.async_copy(
                pos_hbm.at[pl.ds(pbase + c * PC, PC)], pbufs[c % 2],
                psems[c % 2])
            return gh, ph

        handles = [None] * n_chunks
        handles[0] = start_chunk(0)
        for c in range(n_chunks):
            gh, ph = handles[c]
            gh.wait()
            ph.wait()
            if c + 1 < n_chunks:
                handles[c + 1] = start_chunk(c + 1)
            gb = gbufs[c % 2]
            pb = pbufs[c % 2]

            def fma_row(p, _):
                for kv in range(nvec):
                    col = pl.ds(kv * 16, 16)
                    pv = pb[p, col]
                    for b in range(batch):
                        r = b * PC + p
                        gb[r, col] = gb[r, col] * factor + pv
                return _

            lax.fori_loop(0, PC, fma_row, 0)

            for b in range(batch):
                pltpu.sync_copy(
                    gb.at[pl.ds(b * PC, PC)],
                    out_hbm.at[pl.ds(b * seq + pbase + c * PC, PC)])

    return k(xidx, word_table, pos_table)


def kernel(x, word_table, pos_table):
    batch, seq = x.shape
    D = word_table.shape[1]
    p_per_w = seq // _NW
    PC = 16
    n_chunks = p_per_w // PC
    xidx = (x.reshape(batch, _NW, n_chunks, PC)
             .transpose(1, 2, 0, 3)
             .reshape(_NW, n_chunks, batch * PC))
    out, mask_i32 = _sc_fused(xidx, word_table, pos_table, batch, seq)
    mask = (mask_i32.reshape(_NW, n_chunks, batch, PC)
            .transpose(2, 0, 1, 3)
            .reshape(batch, seq)
            .astype(jnp.bool_))
    return out.reshape(batch, seq, D), mask


# speedup vs baseline: 1.3754x; 1.3754x over previous
import functools

import jax
import jax.numpy as jnp
from jax import lax
from jax.experimental import pallas as pl
from jax.experimental.pallas import tpu as pltpu
from jax.experimental.pallas import tpu_sc as plsc

_NC = 2
_NS = 16
_NW = _NC * _NS
_PC = 8


def _sc_fused(xidx, word_table, pos_table, batch, seq):
    V, D = word_table.shape
    B = batch * seq
    p_per_w = seq // _NW
    n_chunks = p_per_w // _PC
    rows = batch * _PC
    factor = float(D) ** 0.5
    nvec = D // 16

    mesh = plsc.VectorSubcoreMesh(core_axis_name="c", subcore_axis_name="s")

    @functools.partial(
        pl.kernel,
        mesh=mesh,
        out_type=[
            jax.ShapeDtypeStruct((B, D), jnp.float32),
            jax.ShapeDtypeStruct((_NW, n_chunks, rows), jnp.int32),
        ],
        scratch_types=[
            pltpu.VMEM((n_chunks, rows), jnp.int32),
            pltpu.VMEM((n_chunks, rows), jnp.int32),
            pltpu.VMEM((rows, D), jnp.float32),
            pltpu.VMEM((rows, D), jnp.float32),
            pltpu.VMEM((rows, D), jnp.float32),
            pltpu.VMEM((rows, D), jnp.float32),
            pltpu.VMEM((_PC, D), jnp.float32),
            pltpu.VMEM((_PC, D), jnp.float32),
            pltpu.VMEM((rows,), jnp.int32),
            pltpu.VMEM((rows,), jnp.int32),
            pltpu.SemaphoreType.DMA,
            pltpu.SemaphoreType.DMA,
            pltpu.SemaphoreType.DMA,
            pltpu.SemaphoreType.DMA,
        ],
    )
    def k(x_hbm, wt_hbm, pos_hbm, out_hbm, mask_hbm,
          idx_v, mask_v, gbuf0, gbuf1, obuf0, obuf1, pbuf0, pbuf1,
          idxb0, idxb1, gsem0, gsem1, psem0, psem1):
        wid = lax.axis_index("s") * _NC + lax.axis_index("c")
        pbase = wid * p_per_w
        gbufs = (gbuf0, gbuf1)
        obufs = (obuf0, obuf1)
        pbufs = (pbuf0, pbuf1)
        gsems = (gsem0, gsem1)
        psems = (psem0, psem1)

        pltpu.sync_copy(x_hbm.at[wid], idx_v)

        for c in range(n_chunks):
            for g in range(rows // 16):
                v = idx_v[c, pl.ds(g * 16, 16)]
                mask_v[c, pl.ds(g * 16, 16)] = jnp.where(
                    v == 0, jnp.full((16,), 1, jnp.int32),
                    jnp.full((16,), 0, jnp.int32))
        pltpu.sync_copy(mask_v, mask_hbm.at[wid])

        idxbs = (idxb0, idxb1)

        def gather_descr(c, d):
            return (
                pltpu.make_async_copy(wt_hbm.at[idxbs[d]], gbufs[d],
                                      gsems[d]),
                pltpu.make_async_copy(pos_hbm.at[pl.ds(pbase + c * _PC, _PC)],
                                      pbufs[d], psems[d]),
            )

        def start_chunk(c, d):
            pltpu.sync_copy(x_hbm.at[wid, c], idxbs[d])
            gh, ph = gather_descr(c, d)
            gh.start()
            ph.start()

        for d in (0, 1):
            start_chunk(d, d)

        def chunk_pair(g, carry):
            for d in (0, 1):
                c = 2 * g + d
                gh, ph = gather_descr(c, d)
                gh.wait()
                ph.wait()

                gb = gbufs[d]
                ob = obufs[d]
                pb = pbufs[d]

                def fma_col(kv, cc):
                    col = pl.ds(kv * 16, 16)
                    for p in range(_PC):
                        pv = pb[p, col]
                        for b in range(batch):
                            r = b * _PC + p
                            ob[r, col] = gb[r, col] * factor + pv
                    return cc

                lax.fori_loop(0, nvec, fma_col, 0)

                @pl.when(g < (n_chunks // 2) - 1)
                def _():
                    start_chunk(c + 2, d)

                for b in range(batch):
                    pltpu.sync_copy(
                        ob.at[pl.ds(b * _PC, _PC)],
                        out_hbm.at[pl.ds(b * seq + pbase + c * _PC, _PC)])
            return carry

        lax.fori_loop(0, n_chunks // 2, chunk_pair, 0)

    return k(xidx, word_table, pos_table)


def kernel(x, word_table, pos_table):
    batch, seq = x.shape
    D = word_table.shape[1]
    p_per_w = seq // _NW
    n_chunks = p_per_w // _PC
    xidx = (x.reshape(batch, _NW, n_chunks, _PC)
             .transpose(1, 2, 0, 3)
             .reshape(_NW, n_chunks, batch * _PC))
    out, mask_i32 = _sc_fused(xidx, word_table, pos_table, batch, seq)
    mask = (mask_i32.reshape(_NW, n_chunks, batch, _PC)
            .transpose(2, 0, 1, 3)
            .reshape(batch, seq)
            .astype(jnp.bool_))
    return out.reshape(batch, seq, D), mask


# speedup vs baseline: 1.6218x; 1.1792x over previous
import functools

import jax
import jax.numpy as jnp
from jax import lax
from jax.experimental import pallas as pl
from jax.experimental.pallas import tpu as pltpu
from jax.experimental.pallas import tpu_sc as plsc

_NC = 2
_NS = 16
_NW = _NC * _NS
_PC = 4


def _sc_fused(xidx, word_table, pos_table, batch, seq):
    V, D = word_table.shape
    B = batch * seq
    p_per_w = seq // _NW
    n_chunks = p_per_w // _PC
    rows = batch * _PC
    factor = float(D) ** 0.5
    nvec = D // 16

    mesh = plsc.VectorSubcoreMesh(core_axis_name="c", subcore_axis_name="s")

    @functools.partial(
        pl.kernel,
        mesh=mesh,
        out_type=[
            jax.ShapeDtypeStruct((B, D), jnp.float32),
            jax.ShapeDtypeStruct((_NW, n_chunks, rows), jnp.int32),
        ],
        scratch_types=[
            pltpu.VMEM((n_chunks, rows), jnp.int32),
            pltpu.VMEM((n_chunks, rows), jnp.int32),
            pltpu.VMEM((rows, D), jnp.float32),
            pltpu.VMEM((rows, D), jnp.float32),
            pltpu.VMEM((rows, D), jnp.float32),
            pltpu.VMEM((rows, D), jnp.float32),
            pltpu.VMEM((_PC, D), jnp.float32),
            pltpu.VMEM((_PC, D), jnp.float32),
            pltpu.SemaphoreType.DMA,
            pltpu.SemaphoreType.DMA,
            pltpu.SemaphoreType.DMA,
            pltpu.SemaphoreType.DMA,
            pltpu.SemaphoreType.DMA,
            pltpu.SemaphoreType.DMA,
        ],
    )
    def k(x_hbm, wt_hbm, pos_hbm, out_hbm, mask_hbm,
          idx_v, mask_v, gbuf0, gbuf1, obuf0, obuf1, pbuf0, pbuf1,
          gsem0, gsem1, psem0, psem1, ssem0, ssem1):
        wid = lax.axis_index("s") * _NC + lax.axis_index("c")
        pbase = wid * p_per_w
        gbufs = (gbuf0, gbuf1)
        obufs = (obuf0, obuf1)
        pbufs = (pbuf0, pbuf1)
        gsems = (gsem0, gsem1)
        psems = (psem0, psem1)
        ssems = (ssem0, ssem1)

        pltpu.sync_copy(x_hbm.at[wid], idx_v)

        def gather_descr(c, d):
            return (
                pltpu.make_async_copy(wt_hbm.at[idx_v.at[c]], gbufs[d],
                                      gsems[d]),
                pltpu.make_async_copy(pos_hbm.at[pl.ds(pbase + c * _PC, _PC)],
                                      pbufs[d], psems[d]),
            )

        def start_chunk(c, d):
            gh, ph = gather_descr(c, d)
            gh.start()
            ph.start()

        for d in (0, 1):
            start_chunk(d, d)

        for c in range(n_chunks):
            for g in range(rows // 16):
                v = idx_v[c, pl.ds(g * 16, 16)]
                mask_v[c, pl.ds(g * 16, 16)] = jnp.where(
                    v == 0, jnp.full((16,), 1, jnp.int32),
                    jnp.full((16,), 0, jnp.int32))
        pltpu.sync_copy(mask_v, mask_hbm.at[wid])

        def store_descrs(c, d):
            return [
                pltpu.make_async_copy(
                    obufs[d].at[pl.ds(b * _PC, _PC)],
                    out_hbm.at[pl.ds(b * seq + pbase + c * _PC, _PC)],
                    ssems[d])
                for b in range(batch)
            ]

        def chunk_pair(g, carry):
            for d in (0, 1):
                c = 2 * g + d
                gh, ph = gather_descr(c, d)
                gh.wait()
                ph.wait()

                @pl.when(g > 0)
                def _():
                    for h in store_descrs(c - 2, d):
                        h.wait()

                gb = gbufs[d]
                ob = obufs[d]
                pb = pbufs[d]

                @plsc.parallel_loop(0, nvec)
                def _(kv):
                    col = pl.ds(kv * 16, 16)
                    for p in range(_PC):
                        pv = pb[p, col]
                        for b in range(batch):
                            r = b * _PC + p
                            ob[r, col] = gb[r, col] * factor + pv

                @pl.when(g < (n_chunks // 2) - 1)
                def _():
                    start_chunk(c + 2, d)

                for h in store_descrs(c, d):
                    h.start()
            return carry

        lax.fori_loop(0, n_chunks // 2, chunk_pair, 0)
        for d in (0, 1):
            for h in store_descrs(n_chunks - 2 + d, d):
                h.wait()

    return k(xidx, word_table, pos_table)


def kernel(x, word_table, pos_table):
    batch, seq = x.shape
    D = word_table.shape[1]
    p_per_w = seq // _NW
    n_chunks = p_per_w // _PC
    xidx = (x.reshape(batch, _NW, n_chunks, _PC)
             .transpose(1, 2, 0, 3)
             .reshape(_NW, n_chunks, batch * _PC))
    out, mask_i32 = _sc_fused(xidx, word_table, pos_table, batch, seq)
    mask = (mask_i32.reshape(_NW, n_chunks, batch, _PC)
            .transpose(2, 0, 1, 3)
            .reshape(batch, seq)
            .astype(jnp.bool_))
    return out.reshape(batch, seq, D), mask


# speedup vs baseline: 1.6543x; 1.0200x over previous
import functools

import jax
import jax.numpy as jnp
from jax import lax
from jax.experimental import pallas as pl
from jax.experimental.pallas import tpu as pltpu
from jax.experimental.pallas import tpu_sc as plsc

_NC = 2
_NS = 16
_NW = _NC * _NS
_PC = 8


def _sc_fused(xidx, word_table, pos_table, batch, seq):
    V, D = word_table.shape
    B = batch * seq
    p_per_w = seq // _NW
    n_chunks = p_per_w // _PC
    rows = batch * _PC
    factor = float(D) ** 0.5
    nvec = D // 16

    mesh = plsc.VectorSubcoreMesh(core_axis_name="c", subcore_axis_name="s")

    @functools.partial(
        pl.kernel,
        mesh=mesh,
        out_type=[
            jax.ShapeDtypeStruct((B, D), jnp.float32),
            jax.ShapeDtypeStruct((_NW, n_chunks, rows), jnp.int32),
        ],
        scratch_types=[
            pltpu.VMEM((n_chunks, rows), jnp.int32),
            pltpu.VMEM((n_chunks, rows), jnp.int32),
            pltpu.VMEM((rows, D), jnp.float32),
            pltpu.VMEM((rows, D), jnp.float32),
            pltpu.VMEM((rows, D), jnp.float32),
            pltpu.VMEM((rows, D), jnp.float32),
            pltpu.VMEM((_PC, D), jnp.float32),
            pltpu.VMEM((_PC, D), jnp.float32),
            pltpu.SemaphoreType.DMA,
            pltpu.SemaphoreType.DMA,
            pltpu.SemaphoreType.DMA,
            pltpu.SemaphoreType.DMA,
            pltpu.SemaphoreType.DMA,
            pltpu.SemaphoreType.DMA,
        ],
    )
    def k(x_hbm, wt_hbm, pos_hbm, out_hbm, mask_hbm,
          idx_v, mask_v, gbuf0, gbuf1, obuf0, obuf1, pbuf0, pbuf1,
          gsem0, gsem1, psem0, psem1, ssem0, ssem1):
        wid = lax.axis_index("s") * _NC + lax.axis_index("c")
        pbase = wid * p_per_w
        gbufs = (gbuf0, gbuf1)
        obufs = (obuf0, obuf1)
        pbufs = (pbuf0, pbuf1)
        gsems = (gsem0, gsem1)
        psems = (psem0, psem1)
        ssems = (ssem0, ssem1)

        pltpu.sync_copy(x_hbm.at[wid], idx_v)

        def gather_descr(c, d):
            return (
                pltpu.make_async_copy(wt_hbm.at[idx_v.at[c]], gbufs[d],
                                      gsems[d]),
                pltpu.make_async_copy(pos_hbm.at[pl.ds(pbase + c * _PC, _PC)],
                                      pbufs[d], psems[d]),
            )

        def start_chunk(c, d):
            gh, ph = gather_descr(c, d)
            gh.start()
            ph.start()

        for d in (0, 1):
            start_chunk(d, d)

        for c in range(n_chunks):
            for g in range(rows // 16):
                v = idx_v[c, pl.ds(g * 16, 16)]
                mask_v[c, pl.ds(g * 16, 16)] = jnp.where(
                    v == 0, jnp.full((16,), 1, jnp.int32),
                    jnp.full((16,), 0, jnp.int32))
        pltpu.sync_copy(mask_v, mask_hbm.at[wid])

        def store_descrs(c, d):
            return [
                pltpu.make_async_copy(
                    obufs[d].at[pl.ds(b * _PC, _PC)],
                    out_hbm.at[pl.ds(b * seq + pbase + c * _PC, _PC)],
                    ssems[d])
                for b in range(batch)
            ]

        def chunk_pair(g, carry):
            for d in (0, 1):
                c = 2 * g + d
                gh, ph = gather_descr(c, d)
                gh.wait()
                ph.wait()

                @pl.when(g > 0)
                def _():
                    for h in store_descrs(c - 2, d):
                        h.wait()

                gb = gbufs[d]
                ob = obufs[d]
                pb = pbufs[d]

                @plsc.parallel_loop(0, nvec)
                def _(kv):
                    col = pl.ds(kv * 16, 16)
                    for p in range(_PC):
                        pv = pb[p, col]
                        for b in range(batch):
                            r = b * _PC + p
                            ob[r, col] = gb[r, col] * factor + pv

                @pl.when(g < (n_chunks // 2) - 1)
                def _():
                    start_chunk(c + 2, d)

                for h in store_descrs(c, d):
                    h.start()
            return carry

        lax.fori_loop(0, n_chunks // 2, chunk_pair, 0)
        for d in (0, 1):
            for h in store_descrs(n_chunks - 2 + d, d):
                h.wait()

    return k(xidx, word_table, pos_table)


def kernel(x, word_table, pos_table):
    batch, seq = x.shape
    D = word_table.shape[1]
    p_per_w = seq // _NW
    n_chunks = p_per_w // _PC
    xidx = (x.reshape(batch, _NW, n_chunks, _PC)
             .transpose(1, 2, 0, 3)
             .reshape(_NW, n_chunks, batch * _PC))
    out, mask_i32 = _sc_fused(xidx, word_table, pos_table, batch, seq)
    mask = (mask_i32.reshape(_NW, n_chunks, batch, _PC)
            .transpose(2, 0, 1, 3)
            .reshape(batch, seq)
            .astype(jnp.bool_))
    return out.reshape(batch, seq, D), mask
